# baseline (speedup 1.0000x reference)
import functools
import math

import jax
import jax.numpy as jnp
from jax import lax
from jax.experimental import pallas as pl
from jax.experimental.pallas import tpu as pltpu
from jax.experimental.pallas import tpu_sc as plsc

D_MODEL = 64
SCALE = math.sqrt(D_MODEL)
NUM_CORES = 2
NUM_SUBCORES = 16
NW = NUM_CORES * NUM_SUBCORES
GROUP = 128
NBUF = 4


def _sc_embed(idx3, table):
    nw, gpw, group = idx3.shape
    b_per_w = gpw * group
    B = nw * b_per_w
    n_rounds = gpw // NBUF
    mesh = plsc.VectorSubcoreMesh(
        core_axis_name="c", subcore_axis_name="s", num_cores=NUM_CORES
    )

    @functools.partial(
        pl.kernel,
        out_type=jax.ShapeDtypeStruct((B, D_MODEL), jnp.float32),
        mesh=mesh,
        scratch_types=[
            pltpu.VMEM((gpw, group), jnp.int32),
            [pltpu.VMEM((group, D_MODEL), jnp.float32) for _ in range(NBUF)],
            [pltpu.VMEM((group, D_MODEL), jnp.float32) for _ in range(NBUF)],
            [pltpu.SemaphoreType.DMA for _ in range(NBUF)],
            [pltpu.SemaphoreType.DMA for _ in range(NBUF)],
        ],
        compiler_params=pltpu.CompilerParams(use_tc_tiling_on_sc=False),
    )
    def k(idx_hbm, table_hbm, out_hbm, idx_v, gbufs, wbufs, gsems, wsems):
        wid = lax.axis_index("s") * NUM_CORES + lax.axis_index("c")
        base = wid * b_per_w
        pltpu.sync_copy(idx_hbm.at[wid], idx_v)

        for b in range(NBUF):
            pltpu.async_copy(table_hbm.at[idx_v.at[b]], gbufs[b], gsems[b])

        def round_body(i, carry):
            for b in range(NBUF):
                gc = i * NBUF + b
                pltpu.make_async_copy(
                    table_hbm.at[idx_v.at[gc]], gbufs[b], gsems[b]
                ).wait()
                @pl.when(i > 0)
                def _wait_write():
                    pltpu.make_async_copy(
                        wbufs[b],
                        out_hbm.at[pl.ds(base, group)],
                        wsems[b],
                    ).wait()

                def row_body(r, c2):
                    for c in range(D_MODEL // 16):
                        sl = pl.ds(c * 16, 16)
                        wbufs[b][r, sl] = gbufs[b][r, sl] * SCALE
                    return c2

                lax.fori_loop(0, group, row_body, 0, unroll=8)

                @pl.when(gc + NBUF < gpw)
                def _rearm():
                    pltpu.async_copy(
                        table_hbm.at[idx_v.at[gc + NBUF]], gbufs[b], gsems[b]
                    )

                pltpu.async_copy(
                    wbufs[b],
                    out_hbm.at[pl.ds(base + gc * group, group)],
                    wsems[b],
                )
            return carry

        lax.fori_loop(0, n_rounds, round_body, 0)

        for b in range(NBUF):
            pltpu.make_async_copy(
                wbufs[b], out_hbm.at[pl.ds(base, group)], wsems[b]
            ).wait()

    return k(idx3, table)


def kernel(x, table):
    S, T = x.shape
    B = S * T
    b_per_w = B // NW
    gpw = b_per_w // GROUP
    idx3 = x.reshape(NW, gpw, GROUP).astype(jnp.int32)
    out = _sc_embed(idx3, table)
    return out.reshape(S, T, D_MODEL)


# speedup vs baseline: 1.0018x; 1.0018x over previous
import functools
import math

import jax
import jax.numpy as jnp
from jax import lax
from jax.experimental import pallas as pl
from jax.experimental.pallas import tpu as pltpu
from jax.experimental.pallas import tpu_sc as plsc

D_MODEL = 64
SCALE = math.sqrt(D_MODEL)
NUM_CORES = 2
NUM_SUBCORES = 16
NW = NUM_CORES * NUM_SUBCORES
NBUF = 4


def _sc_embed(x, table):
    S, T = x.shape
    rows_pw = S // NW
    group = T
    n_groups = rows_pw
    n_rounds = n_groups // NBUF
    mesh = plsc.VectorSubcoreMesh(
        core_axis_name="c", subcore_axis_name="s", num_cores=NUM_CORES
    )

    @functools.partial(
        pl.kernel,
        out_type=jax.ShapeDtypeStruct((S, T, D_MODEL), jnp.float32),
        mesh=mesh,
        scratch_types=[
            pltpu.VMEM((rows_pw, T), jnp.int32),
            [pltpu.VMEM((group, D_MODEL), jnp.float32) for _ in range(NBUF)],
            [pltpu.VMEM((group, D_MODEL), jnp.float32) for _ in range(NBUF)],
            [pltpu.SemaphoreType.DMA for _ in range(NBUF)],
            [pltpu.SemaphoreType.DMA for _ in range(NBUF)],
        ],
        compiler_params=pltpu.CompilerParams(use_tc_tiling_on_sc=False),
    )
    def k(idx_hbm, table_hbm, out_hbm, idx_v, gbufs, wbufs, gsems, wsems):
        wid = lax.axis_index("s") * NUM_CORES + lax.axis_index("c")
        row0 = wid * rows_pw
        pltpu.sync_copy(idx_hbm.at[pl.ds(row0, rows_pw)], idx_v)

        def idx_slice(g):
            return idx_v.at[g]

        def out_slice(g):
            return out_hbm.at[row0 + g]

        for b in range(NBUF):
            pltpu.async_copy(table_hbm.at[idx_slice(b)], gbufs[b], gsems[b])

        def round_body(i, carry):
            for b in range(NBUF):
                g = i * NBUF + b
                pltpu.make_async_copy(
                    table_hbm.at[idx_slice(g)], gbufs[b], gsems[b]
                ).wait()
                @pl.when(i > 0)
                def _wait_write():
                    pltpu.make_async_copy(
                        wbufs[b], out_slice(g), wsems[b]
                    ).wait()

                def row_body(r, c2):
                    for c in range(D_MODEL // 16):
                        sl = pl.ds(c * 16, 16)
                        wbufs[b][r, sl] = gbufs[b][r, sl] * SCALE
                    return c2

                lax.fori_loop(0, group, row_body, 0, unroll=10)

                @pl.when(g + NBUF < n_groups)
                def _rearm():
                    pltpu.async_copy(
                        table_hbm.at[idx_slice(g + NBUF)], gbufs[b], gsems[b]
                    )

                pltpu.async_copy(wbufs[b], out_slice(g), wsems[b])
            return carry

        lax.fori_loop(0, n_rounds, round_body, 0)

        for b in range(NBUF):
            pltpu.make_async_copy(
                wbufs[b], out_hbm.at[row0], wsems[b]
            ).wait()

    return k(x, table)


def kernel(x, table):
    return _sc_embed(x.astype(jnp.int32), table)


# speedup vs baseline: 1.0081x; 1.0062x over previous
import functools
import math

import jax
import jax.numpy as jnp
from jax import lax
from jax.experimental import pallas as pl
from jax.experimental.pallas import tpu as pltpu
from jax.experimental.pallas import tpu_sc as plsc

D_MODEL = 64
SCALE = math.sqrt(D_MODEL)
NUM_CORES = 2
NUM_SUBCORES = 16
NW = NUM_CORES * NUM_SUBCORES
NBUF = 6
LAG = 3


def _sc_embed(x, table):
    S, T = x.shape
    rows_pw = S // NW
    n_groups = rows_pw
    mesh = plsc.VectorSubcoreMesh(
        core_axis_name="c", subcore_axis_name="s", num_cores=NUM_CORES
    )

    @functools.partial(
        pl.kernel,
        out_type=jax.ShapeDtypeStruct((S, T, D_MODEL), jnp.float32),
        mesh=mesh,
        scratch_types=[
            pltpu.VMEM((rows_pw, T), jnp.int32),
            [pltpu.VMEM((T, D_MODEL), jnp.float32) for _ in range(NBUF)],
            [pltpu.SemaphoreType.DMA for _ in range(NBUF)],
            [pltpu.SemaphoreType.DMA for _ in range(NBUF)],
        ],
        compiler_params=pltpu.CompilerParams(use_tc_tiling_on_sc=False),
    )
    def k(idx_hbm, table_hbm, out_hbm, idx_v, gbufs, gsems, wsems):
        wid = lax.axis_index("s") * NUM_CORES + lax.axis_index("c")
        row0 = wid * rows_pw
        pltpu.sync_copy(idx_hbm.at[pl.ds(row0, rows_pw)], idx_v)

        def visit(v, carry):
            @pl.when(v < n_groups)
            def _issue_gather():
                for b in range(NBUF):
                    @pl.when(lax.rem(v, NBUF) == b)
                    def _g():
                        @pl.when(v >= NBUF)
                        def _wait_w():
                            pltpu.make_async_copy(
                                gbufs[b], out_hbm.at[row0], wsems[b]
                            ).wait()
                        pltpu.async_copy(
                            table_hbm.at[idx_v.at[v]], gbufs[b], gsems[b]
                        )

            w = v - LAG
            @pl.when((w >= 0) & (w < n_groups))
            def _issue_write():
                for b in range(NBUF):
                    @pl.when(lax.rem(w, NBUF) == b)
                    def _w():
                        pltpu.make_async_copy(
                            table_hbm.at[idx_v.at[w]], gbufs[b], gsems[b]
                        ).wait()
                        pltpu.async_copy(
                            gbufs[b], out_hbm.at[row0 + w], wsems[b]
                        )
            return carry

        lax.fori_loop(0, n_groups + LAG, visit, 0)

        for b in range(NBUF):
            pltpu.make_async_copy(gbufs[b], out_hbm.at[row0], wsems[b]).wait()

    return k(x, table)


def kernel(x, table):
    return _sc_embed(x.astype(jnp.int32), table * jnp.float32(SCALE))
